# Initial kernel scaffold; baseline (speedup 1.0000x reference)
import jax
import jax.numpy as jnp
from jax.experimental import pallas as pl


def kernel(X, table, pos_weight):
    raise NotImplementedError("write your pallas kernel here")


# speedup vs baseline: 3.9452x; 3.9452x over previous
import functools
import math

import jax
import jax.numpy as jnp
from jax import lax
from jax.experimental import pallas as pl
from jax.experimental.pallas import tpu as pltpu
from jax.experimental.pallas import tpu_sc as plsc

VOCAB = 100000
HIDDEN = 128
B = 1024
L = 200
NC = 2
NS = 16
NW = NC * NS
B_PER_W = B // NW
SCALE = 1.0 / math.sqrt(HIDDEN)
NVH = HIDDEN // 16


def _emb_body(x_hbm, table_hbm, pos_hbm, out_hbm, pos_v, idx_v, buf_v, sem):
    wid = lax.axis_index("s") * NC + lax.axis_index("c")
    base_b = wid * B_PER_W

    pltpu.sync_copy(pos_hbm.at[pl.ds(0, L)], pos_v)

    def scale_body(i, _):
        t = i // NVH
        h = (i % NVH) * 16
        pos_v[t, pl.ds(h, 16)] = pos_v[t, pl.ds(h, 16)] * SCALE
        return _

    lax.fori_loop(0, L * NVH, scale_body, 0, unroll=8)

    def batch_body(j, _):
        row0 = (base_b + j) * L
        pltpu.sync_copy(x_hbm.at[pl.ds(row0, L)], idx_v)
        pltpu.async_copy(table_hbm.at[idx_v], buf_v, sem).wait()

        def add_body(t, _):
            for h in range(NVH):
                sl = pl.ds(h * 16, 16)
                buf_v[t, sl] = buf_v[t, sl] + pos_v[t, sl]
            return _

        lax.fori_loop(0, L, add_body, 0)
        pltpu.sync_copy(buf_v, out_hbm.at[pl.ds(row0, L)])
        return _

    lax.fori_loop(0, B_PER_W, batch_body, 0)


@jax.jit
def _emb(x_flat, table, pos_weight):
    mesh = plsc.VectorSubcoreMesh(core_axis_name="c", subcore_axis_name="s",
                                  num_cores=NC, num_subcores=NS)
    return pl.kernel(
        _emb_body,
        out_type=jax.ShapeDtypeStruct((B * L, HIDDEN), jnp.float32),
        mesh=mesh,
        scratch_types=[
            pltpu.VMEM((L, HIDDEN), jnp.float32),
            pltpu.VMEM((L,), jnp.int32),
            pltpu.VMEM((L, HIDDEN), jnp.float32),
            pltpu.SemaphoreType.DMA,
        ],
    )(x_flat, table, pos_weight)


def kernel(X, table, pos_weight):
    x_flat = X.reshape(B * L).astype(jnp.int32)
    out = _emb(x_flat, table, pos_weight)
    return out.reshape(B, L, HIDDEN)


# speedup vs baseline: 6.2895x; 1.5942x over previous
import math

import jax
import jax.numpy as jnp
from jax import lax
from jax.experimental import pallas as pl
from jax.experimental.pallas import tpu as pltpu
from jax.experimental.pallas import tpu_sc as plsc

VOCAB = 100000
HIDDEN = 128
B = 1024
L = 200
NC = 2
NS = 16
NW = NC * NS
B_PER_W = B // NW
SCALE = 1.0 / math.sqrt(HIDDEN)
NVH = HIDDEN // 16


def _emb_body(x_hbm, table_hbm, pos_hbm, out_hbm,
              pos_v, x_v, buf0, buf1, gsem0, gsem1, osem0, osem1):
    wid = lax.axis_index("s") * NC + lax.axis_index("c")
    base = wid * B_PER_W

    pltpu.sync_copy(x_hbm.at[pl.ds(base * L, B_PER_W * L)], x_v)
    pltpu.sync_copy(pos_hbm.at[pl.ds(0, L)], pos_v)

    def scale_body(t, _):
        for h in range(NVH):
            sl = pl.ds(h * 16, 16)
            pos_v[t, sl] = pos_v[t, sl] * SCALE
        return _

    lax.fori_loop(0, L, scale_body, 0)

    def add_body_for(buf):
        def add_body(t, _):
            for h in range(NVH):
                sl = pl.ds(h * 16, 16)
                buf[t, sl] = buf[t, sl] + pos_v[t, sl]
            return _
        return add_body

    slots = ((buf0, gsem0, osem0), (buf1, gsem1, osem1))
    out_cp = [None, None]

    gather0 = pltpu.async_copy(table_hbm.at[x_v.at[pl.ds(0, L)]], buf0, gsem0)
    gather_cp = [gather0, None]

    for j in range(B_PER_W):
        k = j % 2
        nk = 1 - k
        buf, _, osem = slots[k]
        nbuf, ngsem, _ = slots[nk]
        if j + 1 < B_PER_W:
            if out_cp[nk] is not None:
                out_cp[nk].wait()
            gather_cp[nk] = pltpu.async_copy(
                table_hbm.at[x_v.at[pl.ds((j + 1) * L, L)]], nbuf, ngsem)
        gather_cp[k].wait()
        lax.fori_loop(0, L, add_body_for(buf), 0)
        out_cp[k] = pltpu.async_copy(
            buf, out_hbm.at[pl.ds((base + j) * L, L)], osem)

    out_cp[0].wait()
    out_cp[1].wait()


@jax.jit
def _emb(x_flat, table, pos_weight):
    mesh = plsc.VectorSubcoreMesh(core_axis_name="c", subcore_axis_name="s",
                                  num_cores=NC, num_subcores=NS)
    return pl.kernel(
        _emb_body,
        out_type=jax.ShapeDtypeStruct((B * L, HIDDEN), jnp.float32),
        mesh=mesh,
        scratch_types=[
            pltpu.VMEM((L, HIDDEN), jnp.float32),
            pltpu.VMEM((B_PER_W * L,), jnp.int32),
            pltpu.VMEM((L, HIDDEN), jnp.float32),
            pltpu.VMEM((L, HIDDEN), jnp.float32),
            pltpu.SemaphoreType.DMA,
            pltpu.SemaphoreType.DMA,
            pltpu.SemaphoreType.DMA,
            pltpu.SemaphoreType.DMA,
        ],
    )(x_flat, table, pos_weight)


def kernel(X, table, pos_weight):
    x_flat = X.reshape(B * L).astype(jnp.int32)
    out = _emb(x_flat, table, pos_weight)
    return out.reshape(B, L, HIDDEN)
